# baseline (speedup 1.0000x reference)
import jax
import jax.numpy as jnp
from jax.experimental import pallas as pl

_NW = 8
_BN = 1024


def _router_block(*refs):
    z_refs = refs[:_NW]
    wg = refs[_NW][...]
    gates_ref, idx_ref = refs[_NW + 1], refs[_NW + 2]
    for w in range(_NW):
        logits = jnp.dot(z_refs[w][...], wg,
                         preferred_element_type=jnp.float32)
        lt = logits.T
        ne = lt.shape[0]
        row = jax.lax.broadcasted_iota(jnp.int32, lt.shape, 0)

        m1 = jnp.max(lt, axis=0, keepdims=True)
        i1 = jnp.min(jnp.where(lt == m1, row, ne),
                     axis=0, keepdims=True)
        masked = jnp.where(row == i1, -jnp.inf, lt)
        m2 = jnp.max(masked, axis=0, keepdims=True)
        i2 = jnp.min(jnp.where(masked == m2, row, ne),
                     axis=0, keepdims=True)

        e2 = jnp.exp(m2 - m1)
        g1 = 1.0 / (1.0 + e2)
        gates_ref[w] = jnp.concatenate([g1, 1.0 - g1], axis=0)
        idx_ref[w] = jnp.concatenate([i1, i2], axis=0)


def kernel(z, Wg):
    n, d = z.shape
    e = Wg.shape[1]
    stripe = n // _NW
    steps = stripe // _BN
    in_specs = [
        pl.BlockSpec((_BN, d), lambda i, w=w, s=steps: (i + w * s, 0))
        for w in range(_NW)
    ] + [pl.BlockSpec((d, e), lambda i: (0, 0))]
    gates, idx = pl.pallas_call(
        _router_block,
        grid=(steps,),
        in_specs=in_specs,
        out_specs=[
            pl.BlockSpec((_NW, 2, _BN), lambda i: (0, 0, i)),
            pl.BlockSpec((_NW, 2, _BN), lambda i: (0, 0, i)),
        ],
        out_shape=[
            jax.ShapeDtypeStruct((_NW, 2, stripe), jnp.float32),
            jax.ShapeDtypeStruct((_NW, 2, stripe), jnp.int32),
        ],
    )(*([z] * _NW), Wg)
    gates = jnp.transpose(gates, (0, 2, 1)).reshape(n, 2)
    idx = jnp.transpose(idx, (0, 2, 1)).reshape(n, 2)
    return gates, idx


# speedup vs baseline: 1.0109x; 1.0109x over previous
import jax
import jax.numpy as jnp
from jax.experimental import pallas as pl

_NW = 8
_BN = 256


def _router_block(*refs):
    z_refs = refs[:_NW]
    wg = refs[_NW][...]
    gates_ref, idx_ref = refs[_NW + 1], refs[_NW + 2]
    for w in range(_NW):
        logits = jnp.dot(z_refs[w][...], wg,
                         preferred_element_type=jnp.float32)
        lt = logits.T
        ne = lt.shape[0]
        row = jax.lax.broadcasted_iota(jnp.int32, lt.shape, 0)

        m1 = jnp.max(lt, axis=0, keepdims=True)
        i1 = jnp.min(jnp.where(lt == m1, row, ne),
                     axis=0, keepdims=True)
        masked = jnp.where(row == i1, -jnp.inf, lt)
        m2 = jnp.max(masked, axis=0, keepdims=True)
        i2 = jnp.min(jnp.where(masked == m2, row, ne),
                     axis=0, keepdims=True)

        e2 = jnp.exp(m2 - m1)
        g1 = 1.0 / (1.0 + e2)
        gates_ref[w] = jnp.concatenate([g1, 1.0 - g1], axis=0)
        idx_ref[w] = jnp.concatenate([i1, i2], axis=0)


def kernel(z, Wg):
    n, d = z.shape
    e = Wg.shape[1]
    stripe = n // _NW
    steps = stripe // _BN
    in_specs = [
        pl.BlockSpec((_BN, d), lambda i, w=w, s=steps: (i + w * s, 0))
        for w in range(_NW)
    ] + [pl.BlockSpec((d, e), lambda i: (0, 0))]
    gates, idx = pl.pallas_call(
        _router_block,
        grid=(steps,),
        in_specs=in_specs,
        out_specs=[
            pl.BlockSpec((_NW, 2, _BN), lambda i: (0, 0, i)),
            pl.BlockSpec((_NW, 2, _BN), lambda i: (0, 0, i)),
        ],
        out_shape=[
            jax.ShapeDtypeStruct((_NW, 2, stripe), jnp.float32),
            jax.ShapeDtypeStruct((_NW, 2, stripe), jnp.int32),
        ],
    )(*([z] * _NW), Wg)
    gates = jnp.transpose(gates, (0, 2, 1)).reshape(n, 2)
    idx = jnp.transpose(idx, (0, 2, 1)).reshape(n, 2)
    return gates, idx


# speedup vs baseline: 1.0572x; 1.0457x over previous
import jax
import jax.numpy as jnp
from jax.experimental import pallas as pl

_NW = 32
_BN = 128


def _router_block(*refs):
    z_refs = refs[:_NW]
    wg = refs[_NW][...]
    gates_ref, idx_ref = refs[_NW + 1], refs[_NW + 2]
    for w in range(_NW):
        logits = jnp.dot(z_refs[w][...], wg,
                         preferred_element_type=jnp.float32)
        lt = logits.T
        ne = lt.shape[0]
        row = jax.lax.broadcasted_iota(jnp.int32, lt.shape, 0)

        m1 = jnp.max(lt, axis=0, keepdims=True)
        i1 = jnp.min(jnp.where(lt == m1, row, ne),
                     axis=0, keepdims=True)
        masked = jnp.where(row == i1, -jnp.inf, lt)
        m2 = jnp.max(masked, axis=0, keepdims=True)
        i2 = jnp.min(jnp.where(masked == m2, row, ne),
                     axis=0, keepdims=True)

        e2 = jnp.exp(m2 - m1)
        g1 = 1.0 / (1.0 + e2)
        gates_ref[w] = jnp.concatenate([g1, 1.0 - g1], axis=0)
        idx_ref[w] = jnp.concatenate([i1, i2], axis=0)


def kernel(z, Wg):
    n, d = z.shape
    e = Wg.shape[1]
    stripe = n // _NW
    steps = stripe // _BN
    in_specs = [
        pl.BlockSpec((_BN, d), lambda i, w=w, s=steps: (i + w * s, 0))
        for w in range(_NW)
    ] + [pl.BlockSpec((d, e), lambda i: (0, 0))]
    gates, idx = pl.pallas_call(
        _router_block,
        grid=(steps,),
        in_specs=in_specs,
        out_specs=[
            pl.BlockSpec((_NW, 2, _BN), lambda i: (0, 0, i)),
            pl.BlockSpec((_NW, 2, _BN), lambda i: (0, 0, i)),
        ],
        out_shape=[
            jax.ShapeDtypeStruct((_NW, 2, stripe), jnp.float32),
            jax.ShapeDtypeStruct((_NW, 2, stripe), jnp.int32),
        ],
    )(*([z] * _NW), Wg)
    gates = jnp.transpose(gates, (0, 2, 1)).reshape(n, 2)
    idx = jnp.transpose(idx, (0, 2, 1)).reshape(n, 2)
    return gates, idx


# speedup vs baseline: 1.1414x; 1.0797x over previous
import jax
import jax.numpy as jnp
from jax.experimental import pallas as pl

_NW = 8
_BN = 512


def _router_block(*refs):
    z_refs = refs[:_NW]
    wgt = refs[_NW][...]
    gates_ref, idx_ref = refs[_NW + 1], refs[_NW + 2]
    for w in range(_NW):
        lt = jax.lax.dot_general(
            wgt, z_refs[w][...],
            dimension_numbers=(((1,), (1,)), ((), ())),
            preferred_element_type=jnp.float32)
        ne = lt.shape[0]
        row = jax.lax.broadcasted_iota(jnp.int32, lt.shape, 0)

        m1 = jnp.max(lt, axis=0, keepdims=True)
        i1 = jnp.min(jnp.where(lt == m1, row, ne),
                     axis=0, keepdims=True)
        masked = jnp.where(row == i1, -jnp.inf, lt)
        m2 = jnp.max(masked, axis=0, keepdims=True)
        i2 = jnp.min(jnp.where(masked == m2, row, ne),
                     axis=0, keepdims=True)

        e2 = jnp.exp(m2 - m1)
        g1 = 1.0 / (1.0 + e2)
        gates_ref[w] = jnp.concatenate([g1, 1.0 - g1], axis=0)
        idx_ref[w] = jnp.concatenate([i1, i2], axis=0)


def kernel(z, Wg):
    n, d = z.shape
    e = Wg.shape[1]
    stripe = n // _NW
    steps = stripe // _BN
    in_specs = [
        pl.BlockSpec((_BN, d), lambda i, w=w, s=steps: (i + w * s, 0))
        for w in range(_NW)
    ] + [pl.BlockSpec((e, d), lambda i: (0, 0))]
    gates, idx = pl.pallas_call(
        _router_block,
        grid=(steps,),
        in_specs=in_specs,
        out_specs=[
            pl.BlockSpec((_NW, 2, _BN), lambda i: (0, 0, i)),
            pl.BlockSpec((_NW, 2, _BN), lambda i: (0, 0, i)),
        ],
        out_shape=[
            jax.ShapeDtypeStruct((_NW, 2, stripe), jnp.float32),
            jax.ShapeDtypeStruct((_NW, 2, stripe), jnp.int32),
        ],
    )(*([z] * _NW), Wg.T)
    gates = jnp.transpose(gates, (0, 2, 1)).reshape(n, 2)
    idx = jnp.transpose(idx, (0, 2, 1)).reshape(n, 2)
    return gates, idx
